# baseline (speedup 1.0000x reference)
import jax
import jax.numpy as jnp
from jax import lax
from jax.experimental import pallas as pl
from jax.experimental.pallas import tpu as pltpu
from jax.experimental.pallas import tpu_sc as plsc

_NW = 32
_W = 128
_TCH = 64


def _gather_body(
    table_hbm, ids_hbm, out_hbm, table_sh, idx_v,
    buf0, buf1, buf2, buf3, sg, so0, so1, so2, so3,
):
    s = lax.axis_index("s")
    w = lax.axis_index("c") * 16 + s
    ch = ids_hbm.shape[0]
    d = table_hbm.shape[1]
    tokens = out_hbm.shape[0]
    tok_w = tokens // _NW
    n_chunks = tok_w // _TCH
    tok_base = w * tok_w

    n_sh = table_sh.shape[0]

    @pl.when(s == 0)
    def _():
        pltpu.sync_copy(table_hbm.at[pl.ds(0, n_sh), :], table_sh)

    plsc.subcore_barrier()

    bufs = (buf0, buf1, buf2, buf3)
    sems = (so0, so1, so2, so3)
    nbuf = len(bufs)
    rows_half = idx_v.shape[1]
    chunks_half = rows_half * (_W // _TCH)

    def wb_dst(c, j):
        return out_hbm.at[pl.ds(tok_base + c * _TCH, _TCH), pl.ds(j * d, d)]

    for h in range(n_chunks // chunks_half):
        pltpu.sync_copy(ids_hbm.at[:, w, pl.ds(h * rows_half, rows_half), :],
                        idx_v)

        @pl.loop(0, chunks_half, step=nbuf)
        def _(g):
            for b in range(nbuf):
                buf = bufs[b]
                so = sems[b]
                c = h * chunks_half + g + b

                def idx_vec(j):
                    return idx_v.at[
                        j, (g + b) // (_W // _TCH),
                        pl.ds((b % (_W // _TCH)) * _TCH, _TCH),
                    ]

                @pl.when(c >= nbuf)
                def _():
                    for j in range(ch):
                        pltpu.make_async_copy(
                            buf.at[j], wb_dst(0, j), so
                        ).wait()

                for j in range(ch):
                    pltpu.async_copy(table_sh.at[idx_vec(j)], buf.at[j], sg)
                for j in range(ch):
                    pltpu.make_async_copy(
                        table_sh.at[idx_vec(j)], buf.at[j], sg
                    ).wait()

                for j in range(ch):
                    pltpu.async_copy(buf.at[j], wb_dst(c, j), so)

    for buf, so in zip(bufs, sems):
        for j in range(ch):
            pltpu.make_async_copy(buf.at[j], wb_dst(0, j), so).wait()


def kernel(input_ids, r_table, g_table, b_table):
    b, t, c = input_ids.shape
    v, d = r_table.shape
    tokens = b * t

    table = jnp.concatenate([r_table, g_table, b_table], axis=0)
    offsets = jnp.arange(c, dtype=input_ids.dtype) * v
    ids3 = lax.reshape(
        input_ids, (c, _NW, tokens // _W // _NW, _W), dimensions=(2, 0, 1)
    )
    ids3 = ids3 + offsets[:, None, None, None]

    mesh = plsc.VectorSubcoreMesh(core_axis_name="c", subcore_axis_name="s")
    gather = pl.kernel(
        _gather_body,
        out_type=jax.ShapeDtypeStruct((tokens, c * d), jnp.float32),
        mesh=mesh,
        compiler_params=pltpu.CompilerParams(
            internal_scratch_in_bytes=128 * 1024,
        ),
        scratch_types=[
            pltpu.VMEM_SHARED((c * v, d), jnp.float32),
            pltpu.VMEM((c, tokens // _W // _NW // 2, _W), jnp.int32),
            pltpu.VMEM((c, _TCH, d), jnp.float32),
            pltpu.VMEM((c, _TCH, d), jnp.float32),
            pltpu.VMEM((c, _TCH, d), jnp.float32),
            pltpu.VMEM((c, _TCH, d), jnp.float32),
            pltpu.SemaphoreType.DMA,
            pltpu.SemaphoreType.DMA,
            pltpu.SemaphoreType.DMA,
            pltpu.SemaphoreType.DMA,
            pltpu.SemaphoreType.DMA,
        ],
    )
    out = gather(table, ids3)
    return out.reshape(b, t, c * d)


# speedup vs baseline: 1.0079x; 1.0079x over previous
import jax
import jax.numpy as jnp
from jax import lax
from jax.experimental import pallas as pl
from jax.experimental.pallas import tpu as pltpu
from jax.experimental.pallas import tpu_sc as plsc

_NW = 32
_W = 128
_TCH = 128


def _gather_body(
    table_hbm, ids_hbm, out_hbm, table_sh, idx_v, buf0, buf1, sg, so0, so1
):
    s = lax.axis_index("s")
    w = lax.axis_index("c") * 16 + s
    ch = ids_hbm.shape[0]
    d = table_hbm.shape[1]
    tokens = out_hbm.shape[0]
    tok_w = tokens // _NW
    n_chunks = tok_w // _TCH
    tok_base = w * tok_w

    n_sh = table_sh.shape[0]

    @pl.when(s == 0)
    def _():
        pltpu.sync_copy(table_hbm.at[pl.ds(0, n_sh), :], table_sh)

    plsc.subcore_barrier()

    bufs = (buf0, buf1)
    sems = (so0, so1)
    half = idx_v.shape[1]

    def wb_dst(c, j):
        return out_hbm.at[pl.ds(tok_base + c * _TCH, _TCH), pl.ds(j * d, d)]

    for h in range(n_chunks // half):
        pltpu.sync_copy(ids_hbm.at[:, w, pl.ds(h * half, half), :], idx_v)

        @pl.loop(0, half, step=2)
        def _(g):
            for b in range(2):
                buf = bufs[b]
                so = sems[b]
                c = h * half + g + b

                @pl.when(c >= 2)
                def _():
                    for j in range(ch):
                        pltpu.make_async_copy(
                            buf.at[j], wb_dst(0, j), so
                        ).wait()

                for j in range(ch):
                    pltpu.async_copy(
                        table_sh.at[idx_v.at[j, g + b]], buf.at[j], sg
                    )
                for j in range(ch):
                    pltpu.make_async_copy(
                        table_sh.at[idx_v.at[j, g + b]], buf.at[j], sg
                    ).wait()

                for j in range(ch):
                    pltpu.async_copy(buf.at[j], wb_dst(c, j), so)

    for buf, so in ((buf0, so0), (buf1, so1)):
        for j in range(ch):
            pltpu.make_async_copy(buf.at[j], wb_dst(0, j), so).wait()


def kernel(input_ids, r_table, g_table, b_table):
    b, t, c = input_ids.shape
    v, d = r_table.shape
    tokens = b * t

    table = jnp.concatenate([r_table, g_table, b_table], axis=0)
    offsets = jnp.arange(c, dtype=input_ids.dtype) * v
    ids3 = lax.reshape(
        input_ids, (c, _NW, tokens // _W // _NW, _W), dimensions=(2, 0, 1)
    )
    ids3 = ids3 + offsets[:, None, None, None]

    mesh = plsc.VectorSubcoreMesh(core_axis_name="c", subcore_axis_name="s")
    gather = pl.kernel(
        _gather_body,
        out_type=jax.ShapeDtypeStruct((tokens, c * d), jnp.float32),
        mesh=mesh,
        compiler_params=pltpu.CompilerParams(
            internal_scratch_in_bytes=128 * 1024,
        ),
        scratch_types=[
            pltpu.VMEM_SHARED((c * v, d), jnp.float32),
            pltpu.VMEM((c, tokens // _W // _NW // 2, _W), jnp.int32),
            pltpu.VMEM((c, _TCH, d), jnp.float32),
            pltpu.VMEM((c, _TCH, d), jnp.float32),
            pltpu.SemaphoreType.DMA,
            pltpu.SemaphoreType.DMA,
            pltpu.SemaphoreType.DMA,
        ],
    )
    out = gather(table, ids3)
    return out.reshape(b, t, c * d)
